# Initial kernel scaffold; baseline (speedup 1.0000x reference)
import jax
import jax.numpy as jnp
from jax.experimental import pallas as pl


def kernel(user_ids, movie_history, actions, user_table, movie_table, action_table):
    raise NotImplementedError("write your pallas kernel here")


# speedup vs baseline: 4.2417x; 4.2417x over previous
import functools

import jax
import jax.numpy as jnp
from jax import lax
from jax.experimental import pallas as pl
from jax.experimental.pallas import tpu as pltpu
from jax.experimental.pallas import tpu_sc as plsc

B = 4096
HIST = 50
D = 32
ROWS_PER_B = HIST + 2
NC = 2
NS = 16
NW = NC * NS
B_PER_W = B // NW
MCHUNKS = B_PER_W * HIST // 128


def _sc_body(user_table, movie_table, action_table, uidx, aidx, midx3, ddx3,
             out, uidx_v, aidx_v, midx_v, ddx_v, didx_u, didx_a,
             ubuf, abuf, mbuf, sem, sem2):
  wid = lax.axis_index("s") * NC + lax.axis_index("c")
  base = wid * B_PER_W

  pltpu.sync_copy(uidx.at[pl.ds(base, B_PER_W)], uidx_v)
  pltpu.sync_copy(aidx.at[pl.ds(base, B_PER_W)], aidx_v)
  pltpu.sync_copy(midx3.at[wid], midx_v)
  pltpu.sync_copy(ddx3.at[wid], ddx_v)

  for i in range(B_PER_W // 16):
    lane = lax.iota(jnp.int32, 16)
    row = (base + i * 16 + lane) * ROWS_PER_B
    didx_u[pl.ds(i * 16, 16)] = row
    didx_a[pl.ds(i * 16, 16)] = row + (ROWS_PER_B - 1)

  pltpu.async_copy(user_table.at[uidx_v], ubuf, sem).wait()
  pltpu.async_copy(ubuf, out.at[didx_u], sem).wait()
  pltpu.async_copy(action_table.at[aidx_v], abuf, sem2).wait()
  pltpu.async_copy(abuf, out.at[didx_a], sem2).wait()

  def body(k, carry):
    pltpu.async_copy(movie_table.at[midx_v.at[k]], mbuf, sem).wait()
    pltpu.async_copy(mbuf, out.at[ddx_v.at[k]], sem).wait()
    return carry

  lax.fori_loop(0, MCHUNKS, body, 0)


@jax.jit
def _sc_embed(user_table, movie_table, action_table, uidx, aidx, midx3, ddx3):
  mesh = plsc.VectorSubcoreMesh(core_axis_name="c", subcore_axis_name="s")
  return pl.kernel(
      _sc_body,
      out_type=jax.ShapeDtypeStruct((B * ROWS_PER_B, D), jnp.float32),
      mesh=mesh,
      compiler_params=pltpu.CompilerParams(use_tc_tiling_on_sc=False),
      scratch_types=[
          pltpu.VMEM((B_PER_W,), jnp.int32),
          pltpu.VMEM((B_PER_W,), jnp.int32),
          pltpu.VMEM((MCHUNKS, 128), jnp.int32),
          pltpu.VMEM((MCHUNKS, 128), jnp.int32),
          pltpu.VMEM((B_PER_W,), jnp.int32),
          pltpu.VMEM((B_PER_W,), jnp.int32),
          pltpu.VMEM((B_PER_W, D), jnp.float32),
          pltpu.VMEM((B_PER_W, D), jnp.float32),
          pltpu.VMEM((128, D), jnp.float32),
          pltpu.SemaphoreType.DMA,
          pltpu.SemaphoreType.DMA,
      ],
  )(user_table, movie_table, action_table, uidx, aidx, midx3, ddx3)


def kernel(user_ids, movie_history, actions, user_table, movie_table,
           action_table):
  uidx = user_ids.reshape(B).astype(jnp.int32)
  aidx = actions.reshape(B).astype(jnp.int32)
  midx3 = movie_history.astype(jnp.int32).reshape(NW, MCHUNKS, 128)
  n = jnp.arange(B * HIST, dtype=jnp.int32)
  ddx3 = ((n // HIST) * ROWS_PER_B + 1 + n % HIST).reshape(NW, MCHUNKS, 128)
  out = _sc_embed(user_table, movie_table, action_table, uidx, aidx, midx3,
                  ddx3)
  return out.reshape(B, ROWS_PER_B * D)


# speedup vs baseline: 6.0823x; 1.4339x over previous
import jax
import jax.numpy as jnp
from jax import lax
from jax.experimental import pallas as pl
from jax.experimental.pallas import tpu as pltpu
from jax.experimental.pallas import tpu_sc as plsc

B = 4096
HIST = 50
D = 32
ROWS_PER_B = HIST + 2
NC = 2
NS = 16
NW = NC * NS
B_PER_W = B // NW
CHUNKS = B_PER_W * ROWS_PER_B // 128
GROUP = 13
GROUP_ROWS = GROUP * 128
TILE_ROWS = B_PER_W * ROWS_PER_B


def _movie_body(movie_table, mh, pp, out,
                mh_v, pp_v, cidx_v, bufA, bufB,
                isem, gsemA, gsemB, wsem0, wsem1):
  wid = lax.axis_index("s") * NC + lax.axis_index("c")
  base = wid * B_PER_W
  out_base = wid * TILE_ROWS
  bufs = (bufA, bufB)
  gsems = (gsemA, gsemB)
  wsems = (wsem0, wsem1)

  idx_pairs = (
      (mh.at[pl.ds(base, B_PER_W)], mh_v),
      (pp.at[:], pp_v),
  )
  for src, dst in idx_pairs:
    pltpu.async_copy(src, dst, isem)
  for src, dst in idx_pairs:
    pltpu.make_async_copy(src, dst, isem).wait()

  for r in range(CHUNKS):
    for g in range(8):
      sl = pl.ds(g * 16, 16)
      v = pp_v[r, sl]
      ri = lax.shift_right_logical(v, 6)
      ci = lax.bitwise_and(v, 63)
      cidx_v[r, sl] = plsc.load_gather(mh_v, [ri, ci])

  def fire_group(g):
    buf, sem = bufs[g % 2], gsems[g % 2]

    def body(c, carry):
      pltpu.async_copy(movie_table.at[cidx_v.at[g * GROUP + c]],
                       buf.at[pl.ds(c * 128, 128)], sem)
      return carry

    lax.fori_loop(0, GROUP, body, 0)

  def drain_group(g):
    pltpu.make_async_copy(movie_table.at[pl.ds(0, GROUP_ROWS)], bufs[g % 2],
                          gsems[g % 2]).wait()

  def w_desc(g):
    return pltpu.make_async_copy(
        bufs[g % 2], out.at[pl.ds(out_base + g * GROUP_ROWS, GROUP_ROWS)],
        wsems[g % 2])

  fire_group(0)
  fire_group(1)
  drain_group(0)
  w_desc(0).start()
  drain_group(1)
  w_desc(1).start()
  w_desc(0).wait()
  fire_group(2)
  drain_group(2)
  w_desc(2).start()
  w_desc(1).wait()
  fire_group(3)
  drain_group(3)
  w_desc(3).start()
  w_desc(2).wait()
  w_desc(3).wait()


def _patch_body(user_table, action_table, uidx, aidx, udst, adst, out_ref,
                uidx_v, aidx_v, udst_v, adst_v, ubuf, abuf,
                isem, usem, asem):
  wid = lax.axis_index("s") * NC + lax.axis_index("c")
  base = wid * B_PER_W

  idx_pairs = (
      (uidx.at[pl.ds(base, B_PER_W)], uidx_v),
      (aidx.at[pl.ds(base, B_PER_W)], aidx_v),
      (udst.at[pl.ds(base, B_PER_W)], udst_v),
      (adst.at[pl.ds(base, B_PER_W)], adst_v),
  )
  for src, dst in idx_pairs:
    pltpu.async_copy(src, dst, isem)
  for src, dst in idx_pairs:
    pltpu.make_async_copy(src, dst, isem).wait()

  pltpu.async_copy(user_table.at[uidx_v], ubuf, usem)
  pltpu.async_copy(action_table.at[aidx_v], abuf, asem)
  pltpu.make_async_copy(user_table.at[uidx_v], ubuf, usem).wait()
  pltpu.async_copy(ubuf, out_ref.at[udst_v], usem)
  pltpu.make_async_copy(action_table.at[aidx_v], abuf, asem).wait()
  pltpu.async_copy(abuf, out_ref.at[adst_v], asem)
  pltpu.make_async_copy(ubuf, out_ref.at[udst_v], usem).wait()
  pltpu.make_async_copy(abuf, out_ref.at[adst_v], asem).wait()


@jax.jit
def _sc_embed(user_table, movie_table, action_table, uidx, aidx, mh):
  mesh = plsc.VectorSubcoreMesh(core_axis_name="c", subcore_axis_name="s")

  p = jnp.arange(CHUNKS * 128, dtype=jnp.int32)
  q, t = p // 4, p % 4
  s = q % 8
  pj = q // 8
  il, jj = pj // GROUP, pj % GROUP
  pp = ((il * 8 + s) * 64
        + jnp.clip(4 * jj + t - 1, 0, HIST - 1)).reshape(CHUNKS, 128)

  bb = jnp.arange(B, dtype=jnp.int32)
  rowbase = ((bb // 8) * 104 + bb % 8) * 4
  udst = rowbase
  adst = rowbase + 387

  movie_out = pl.kernel(
      _movie_body,
      out_type=jax.ShapeDtypeStruct((B * ROWS_PER_B, D), jnp.float32),
      mesh=mesh,
      compiler_params=pltpu.CompilerParams(
          use_tc_tiling_on_sc=False, needs_layout_passes=False),
      scratch_types=[
          pltpu.VMEM((B_PER_W, HIST), jnp.int32),
          pltpu.VMEM((CHUNKS, 128), jnp.int32),
          pltpu.VMEM((CHUNKS, 128), jnp.int32),
          pltpu.VMEM((GROUP_ROWS, D), jnp.float32),
          pltpu.VMEM((GROUP_ROWS, D), jnp.float32),
          pltpu.SemaphoreType.DMA,
          pltpu.SemaphoreType.DMA,
          pltpu.SemaphoreType.DMA,
          pltpu.SemaphoreType.DMA,
          pltpu.SemaphoreType.DMA,
      ],
  )(movie_table, mh, pp)

  out_ref = jax.new_ref(movie_out)
  pl.kernel(
      _patch_body,
      out_type=(),
      mesh=mesh,
      compiler_params=pltpu.CompilerParams(use_tc_tiling_on_sc=False),
      scratch_types=[
          pltpu.VMEM((B_PER_W,), jnp.int32),
          pltpu.VMEM((B_PER_W,), jnp.int32),
          pltpu.VMEM((B_PER_W,), jnp.int32),
          pltpu.VMEM((B_PER_W,), jnp.int32),
          pltpu.VMEM((B_PER_W, D), jnp.float32),
          pltpu.VMEM((B_PER_W, D), jnp.float32),
          pltpu.SemaphoreType.DMA,
          pltpu.SemaphoreType.DMA,
          pltpu.SemaphoreType.DMA,
      ],
  )(user_table, action_table, uidx, aidx, udst, adst, out_ref)
  return out_ref[...]


def kernel(user_ids, movie_history, actions, user_table, movie_table,
           action_table):
  uidx = user_ids.reshape(B).astype(jnp.int32)
  aidx = actions.reshape(B).astype(jnp.int32)
  mh = movie_history.astype(jnp.int32)

  out = _sc_embed(user_table, movie_table, action_table, uidx, aidx, mh)
  return (out.reshape(B // 8, 13, 8, 128)
          .transpose(0, 2, 1, 3)
          .reshape(B, ROWS_PER_B * D))
